# baseline (speedup 1.0000x reference)
import jax
import jax.numpy as jnp
from jax import lax
from jax.experimental import pallas as pl
from jax.experimental.pallas import tpu as pltpu
from jax.experimental.pallas import tpu_sc as plsc

N_ROWS = 16384
N_COLS = 512
N_IDX = 128
LANES = 16

SC_ROWS = 4096
TC_ROWS = N_ROWS - SC_ROWS
TC_BLK = 2048

NUM_CORES = 2
NUM_SUBCORES = 16
NUM_WORKERS = NUM_CORES * NUM_SUBCORES
ROWS_PER_WORKER = SC_ROWS // NUM_WORKERS
CHUNK_ROWS = 128
NUM_CHUNKS = ROWS_PER_WORKER // CHUNK_ROWS
IDX_GROUPS = N_IDX // LANES
NBUF = 1


def _body(in_hbm, idx_hbm, out_hbm, idx_v, in_bufs, out_bufs, sem_idx, sem_is, sem_os):
    wid = lax.axis_index("s") * NUM_CORES + lax.axis_index("c")
    row0 = wid * ROWS_PER_WORKER

    idx_copy = pltpu.async_copy(idx_hbm, idx_v, sem_idx)

    def in_slice(ch):
        return in_hbm.at[pl.ds(row0 + ch * CHUNK_ROWS, CHUNK_ROWS)]

    def out_slice(ch):
        return out_hbm.at[pl.ds(row0 + ch * CHUNK_ROWS, CHUNK_ROWS)]

    for ch in range(NBUF - 1):
        pltpu.async_copy(in_slice(ch), in_bufs[ch], sem_is[ch])

    idx_copy.wait()
    idx_groups = [idx_v[pl.ds(g * LANES, LANES)] for g in range(IDX_GROUPS)]

    @pl.loop(0, NUM_CHUNKS, step=NBUF)
    def _chunk(ch0):
        for b in range(NBUF):
            ch = ch0 + b

            @pl.when(ch + NBUF - 1 < NUM_CHUNKS)
            def _():
                nb = (b + NBUF - 1) % NBUF
                pltpu.async_copy(in_slice(ch + NBUF - 1), in_bufs[nb], sem_is[nb])

            pltpu.make_async_copy(in_slice(0), in_bufs[b], sem_is[b]).wait()

            @pl.when(ch >= NBUF)
            def _():
                pltpu.make_async_copy(out_bufs[b], out_slice(0), sem_os[b]).wait()

            @pl.loop(0, CHUNK_ROWS, unroll=2)
            def _gather(r):
                rowv = jnp.full((LANES,), r, dtype=jnp.int32)
                for g in range(IDX_GROUPS):
                    out_bufs[b][r, pl.ds(g * LANES, LANES)] = plsc.load_gather(
                        in_bufs[b], [rowv, idx_groups[g]]
                    )

            pltpu.async_copy(out_bufs[b], out_slice(ch), sem_os[b])

    for b in range(NBUF):
        pltpu.make_async_copy(out_bufs[b], out_slice(0), sem_os[b]).wait()


def _tc_body(idx_ref, in_ref, out_ref):
    col = lax.broadcasted_iota(jnp.int32, (N_COLS, N_IDX), 0)
    onehot = (col == idx_ref[0, :][None, :]).astype(jnp.float32)
    out_ref[...] = jnp.dot(
        in_ref[...], onehot, preferred_element_type=jnp.float32
    )


def _tc_gather(input, indexes2d):
    return pl.pallas_call(
        _tc_body,
        grid=(TC_ROWS // TC_BLK,),
        in_specs=[
            pl.BlockSpec((1, N_IDX), lambda i: (0, 0)),
            pl.BlockSpec((TC_BLK, N_COLS), lambda i: (i + SC_ROWS // TC_BLK, 0)),
        ],
        out_specs=pl.BlockSpec((TC_BLK, N_IDX), lambda i: (i + SC_ROWS // TC_BLK, 0)),
        out_shape=jax.ShapeDtypeStruct((N_ROWS, N_IDX), jnp.float32),
    )(indexes2d, input)


@jax.jit
def kernel(input, indexes):
    mesh = plsc.VectorSubcoreMesh(
        core_axis_name="c",
        subcore_axis_name="s",
        num_cores=NUM_CORES,
        num_subcores=NUM_SUBCORES,
    )
    run = pl.kernel(
        _body,
        out_type=jax.ShapeDtypeStruct((SC_ROWS, N_IDX), jnp.float32),
        mesh=mesh,
        scratch_types=[
            pltpu.VMEM((N_IDX,), jnp.int32),
            [pltpu.VMEM((CHUNK_ROWS, N_COLS), jnp.float32) for _ in range(NBUF)],
            [pltpu.VMEM((CHUNK_ROWS, N_IDX), jnp.float32) for _ in range(NBUF)],
            pltpu.SemaphoreType.DMA,
            [pltpu.SemaphoreType.DMA for _ in range(NBUF)],
            [pltpu.SemaphoreType.DMA for _ in range(NBUF)],
        ],
        compiler_params=pltpu.CompilerParams(
            use_tc_tiling_on_sc=True, needs_layout_passes=False
        ),
    )
    sc_out = run(input, indexes)
    tc_out = _tc_gather(input, indexes.reshape(1, N_IDX))
    return jax.lax.dynamic_update_slice(tc_out, sc_out, (0, 0))


# speedup vs baseline: 1.0362x; 1.0362x over previous
import jax
import jax.numpy as jnp
from jax import lax
from jax.experimental import pallas as pl
from jax.experimental.pallas import tpu as pltpu
from jax.experimental.pallas import tpu_sc as plsc

N_ROWS = 16384
N_COLS = 512
N_IDX = 128
LANES = 16

SC_ROWS = 4096
TC_ROWS = N_ROWS - SC_ROWS
TC_BLK = 2048

NUM_CORES = 2
NUM_SUBCORES = 16
NUM_WORKERS = NUM_CORES * NUM_SUBCORES
ROWS_PER_WORKER = SC_ROWS // NUM_WORKERS
CHUNK_ROWS = 64
NUM_CHUNKS = ROWS_PER_WORKER // CHUNK_ROWS
IDX_GROUPS = N_IDX // LANES
NBUF = 2


def _body(in_hbm, idx_hbm, out_hbm, idx_v, in_bufs, out_bufs, sem_idx, sem_is, sem_os):
    wid = lax.axis_index("s") * NUM_CORES + lax.axis_index("c")
    row0 = wid * ROWS_PER_WORKER

    idx_copy = pltpu.async_copy(idx_hbm, idx_v, sem_idx)

    def in_slice(ch):
        return in_hbm.at[pl.ds(row0 + ch * CHUNK_ROWS, CHUNK_ROWS)]

    def out_slice(ch):
        return out_hbm.at[pl.ds(row0 + ch * CHUNK_ROWS, CHUNK_ROWS)]

    for ch in range(NBUF - 1):
        pltpu.async_copy(in_slice(ch), in_bufs[ch], sem_is[ch])

    idx_copy.wait()
    idx_groups = [idx_v[pl.ds(g * LANES, LANES)] for g in range(IDX_GROUPS)]

    @pl.loop(0, NUM_CHUNKS, step=NBUF)
    def _chunk(ch0):
        for b in range(NBUF):
            ch = ch0 + b

            @pl.when(ch + NBUF - 1 < NUM_CHUNKS)
            def _():
                nb = (b + NBUF - 1) % NBUF
                pltpu.async_copy(in_slice(ch + NBUF - 1), in_bufs[nb], sem_is[nb])

            pltpu.make_async_copy(in_slice(0), in_bufs[b], sem_is[b]).wait()

            @pl.when(ch >= NBUF)
            def _():
                pltpu.make_async_copy(out_bufs[b], out_slice(0), sem_os[b]).wait()

            @pl.loop(0, CHUNK_ROWS, unroll=2)
            def _gather(r):
                rowv = jnp.full((LANES,), r, dtype=jnp.int32)
                for g in range(IDX_GROUPS):
                    out_bufs[b][r, pl.ds(g * LANES, LANES)] = plsc.load_gather(
                        in_bufs[b], [rowv, idx_groups[g]]
                    )

            pltpu.async_copy(out_bufs[b], out_slice(ch), sem_os[b])

    for b in range(NBUF):
        pltpu.make_async_copy(out_bufs[b], out_slice(0), sem_os[b]).wait()


def _tc_body(idx_ref, in_ref, out_ref):
    col = lax.broadcasted_iota(jnp.int32, (N_COLS, N_IDX), 0)
    onehot = (col == idx_ref[0, :][None, :]).astype(jnp.float32)
    out_ref[...] = jnp.dot(
        in_ref[...], onehot, preferred_element_type=jnp.float32
    )


def _tc_gather(input, indexes2d):
    return pl.pallas_call(
        _tc_body,
        grid=(TC_ROWS // TC_BLK,),
        in_specs=[
            pl.BlockSpec((1, N_IDX), lambda i: (0, 0)),
            pl.BlockSpec((TC_BLK, N_COLS), lambda i: (i + SC_ROWS // TC_BLK, 0)),
        ],
        out_specs=pl.BlockSpec((TC_BLK, N_IDX), lambda i: (i + SC_ROWS // TC_BLK, 0)),
        out_shape=jax.ShapeDtypeStruct((N_ROWS, N_IDX), jnp.float32),
    )(indexes2d, input)


@jax.jit
def kernel(input, indexes):
    mesh = plsc.VectorSubcoreMesh(
        core_axis_name="c",
        subcore_axis_name="s",
        num_cores=NUM_CORES,
        num_subcores=NUM_SUBCORES,
    )
    run = pl.kernel(
        _body,
        out_type=jax.ShapeDtypeStruct((SC_ROWS, N_IDX), jnp.float32),
        mesh=mesh,
        scratch_types=[
            pltpu.VMEM((N_IDX,), jnp.int32),
            [pltpu.VMEM((CHUNK_ROWS, N_COLS), jnp.float32) for _ in range(NBUF)],
            [pltpu.VMEM((CHUNK_ROWS, N_IDX), jnp.float32) for _ in range(NBUF)],
            pltpu.SemaphoreType.DMA,
            [pltpu.SemaphoreType.DMA for _ in range(NBUF)],
            [pltpu.SemaphoreType.DMA for _ in range(NBUF)],
        ],
        compiler_params=pltpu.CompilerParams(
            use_tc_tiling_on_sc=True, needs_layout_passes=False
        ),
    )
    sc_out = run(input, indexes)
    tc_out = _tc_gather(input, indexes.reshape(1, N_IDX))
    return jax.lax.dynamic_update_slice(tc_out, sc_out, (0, 0))
